# Initial kernel scaffold; baseline (speedup 1.0000x reference)
import jax
import jax.numpy as jnp
from jax.experimental import pallas as pl


def kernel(positions, pe):
    raise NotImplementedError("write your pallas kernel here")


# speedup vs baseline: 5.7558x; 5.7558x over previous
import functools

import jax
import jax.numpy as jnp
from jax import lax
from jax.experimental import pallas as pl
from jax.experimental.pallas import tpu as pltpu
from jax.experimental.pallas import tpu_sc as plsc

EMB_D = 128
CHUNK = 128


def _make_sc_gather(n_rows, d):
    info = plsc.get_sparse_core_info()
    nc, ns = info.num_cores, info.num_subcores
    nw = nc * ns
    per_w = n_rows // nw
    n_chunks = per_w // CHUNK
    idx_rows = per_w // CHUNK

    mesh = plsc.VectorSubcoreMesh(core_axis_name="c", subcore_axis_name="s")

    @functools.partial(
        pl.kernel,
        mesh=mesh,
        out_type=jax.ShapeDtypeStruct((n_rows, d), jnp.float32),
        scratch_types=[
            pltpu.VMEM((idx_rows, CHUNK), jnp.int32),
            pltpu.VMEM((CHUNK, d), jnp.float32),
            pltpu.SemaphoreType.DMA,
        ],
    )
    def gather_kernel(table_hbm, idx_hbm, out_hbm, idx_v, rows_v, sem):
        wid = lax.axis_index("s") * nc + lax.axis_index("c")
        base = wid * per_w
        pltpu.sync_copy(idx_hbm.at[pl.ds(wid * idx_rows, idx_rows)], idx_v)

        def body(j, _):
            pltpu.async_copy(table_hbm.at[idx_v.at[j]], rows_v, sem).wait()
            pltpu.sync_copy(rows_v, out_hbm.at[pl.ds(base + j * CHUNK, CHUNK)])
            return _

        lax.fori_loop(0, n_chunks, body, None)

    return gather_kernel


def kernel(positions, pe):
    b, s = positions.shape
    n_rows = b * s
    idx2d = positions.reshape(n_rows // CHUNK, CHUNK)
    fn = _make_sc_gather(n_rows, pe.shape[1])
    out = fn(pe, idx2d)
    return out.reshape(b, s, pe.shape[1])


# speedup vs baseline: 7.9010x; 1.3727x over previous
import functools

import jax
import jax.numpy as jnp
from jax import lax
from jax.experimental import pallas as pl
from jax.experimental.pallas import tpu as pltpu
from jax.experimental.pallas import tpu_sc as plsc

EMB_D = 128
CHUNK = 128
NBUF = 4
K_AHEAD = 2


def _make_sc_gather(n_rows, d):
    info = plsc.get_sparse_core_info()
    nc, ns = info.num_cores, info.num_subcores
    nw = nc * ns
    per_w = n_rows // nw
    n_chunks = per_w // CHUNK
    idx_rows = per_w // CHUNK
    n_outer = n_chunks // NBUF

    mesh = plsc.VectorSubcoreMesh(core_axis_name="c", subcore_axis_name="s")

    @functools.partial(
        pl.kernel,
        mesh=mesh,
        out_type=jax.ShapeDtypeStruct((n_rows, d), jnp.float32),
        scratch_types=[
            pltpu.VMEM((idx_rows, CHUNK), jnp.int32),
            pltpu.VMEM((NBUF, CHUNK, d), jnp.float32),
            pltpu.SemaphoreType.DMA((NBUF,)),
            pltpu.SemaphoreType.DMA((NBUF,)),
        ],
    )
    def gather_kernel(table_hbm, idx_hbm, out_hbm, idx_v, bufs, gsem, osem):
        wid = lax.axis_index("s") * nc + lax.axis_index("c")
        base = wid * per_w
        pltpu.sync_copy(idx_hbm.at[pl.ds(wid * idx_rows, idx_rows)], idx_v)

        def start_gather(g, b):
            pltpu.async_copy(table_hbm.at[idx_v.at[g]], bufs.at[b], gsem.at[b])

        def chunk(g, b, osem_wait, issue):
            b2 = (b + K_AHEAD) % NBUF
            if issue:
                if osem_wait:
                    pltpu.make_async_copy(
                        bufs.at[b2], out_hbm.at[pl.ds(base, CHUNK)], osem.at[b2]
                    ).wait()
                start_gather(g + K_AHEAD, b2)
            pltpu.make_async_copy(
                table_hbm.at[idx_v.at[g]], bufs.at[b], gsem.at[b]
            ).wait()
            pltpu.async_copy(
                bufs.at[b], out_hbm.at[pl.ds(base + g * CHUNK, CHUNK)], osem.at[b]
            )

        start_gather(0, 0)
        start_gather(1, 1)
        chunk(0, 0, osem_wait=False, issue=True)
        chunk(1, 1, osem_wait=False, issue=True)
        chunk(2, 2, osem_wait=True, issue=True)
        chunk(3, 3, osem_wait=True, issue=True)

        def outer(o, _):
            g0 = o * NBUF
            for b in range(NBUF):
                chunk(g0 + b, b, osem_wait=True, issue=True)
            return _

        lax.fori_loop(1, n_outer - 1, outer, None)

        g0 = (n_outer - 1) * NBUF
        chunk(g0 + 0, 0, osem_wait=True, issue=True)
        chunk(g0 + 1, 1, osem_wait=True, issue=True)
        chunk(g0 + 2, 2, osem_wait=False, issue=False)
        chunk(g0 + 3, 3, osem_wait=False, issue=False)

        for b in range(NBUF):
            pltpu.make_async_copy(
                bufs.at[b], out_hbm.at[pl.ds(base, CHUNK)], osem.at[b]
            ).wait()

    return gather_kernel


def kernel(positions, pe):
    b, s = positions.shape
    n_rows = b * s
    idx2d = positions.reshape(n_rows // CHUNK, CHUNK)
    fn = _make_sc_gather(n_rows, pe.shape[1])
    out = fn(pe, idx2d)
    return out.reshape(b, s, pe.shape[1])
